# baseline (speedup 1.0000x reference)
import functools

import jax
import jax.numpy as jnp
from jax.experimental import pallas as pl
from jax.experimental.pallas import tpu as pltpu

N_PAD = 10240
NB = 1024
EB = 2000
D = 256


def _mm_bias_body(x_ref, w_ref, b_ref, o_ref):
    o_ref[...] = (
        jnp.dot(x_ref[...], w_ref[...], preferred_element_type=jnp.float32)
        + b_ref[0:1, :]
    )


def _matmul_bias(x, w, b, blk):
    n, k = x.shape
    m = w.shape[1]
    b2 = jnp.broadcast_to(b.reshape(1, m), (8, m))
    return pl.pallas_call(
        _mm_bias_body,
        grid=(n // blk,),
        in_specs=[
            pl.BlockSpec((blk, k), lambda i: (i, 0)),
            pl.BlockSpec((k, m), lambda i: (0, 0)),
            pl.BlockSpec((8, m), lambda i: (0, 0)),
        ],
        out_specs=pl.BlockSpec((blk, m), lambda i: (i, 0)),
        out_shape=jax.ShapeDtypeStruct((n, m), jnp.float32),
    )(x, w, b2)


def _agg_body(s_ref, q_ref, mx_ref, cnt_ref, w2_ref, b2_ref, nm_ref, out_ref, sums_ref):
    i = pl.program_id(0)
    inv = 1.0 / cnt_ref[...]
    mean = s_ref[...] * inv
    sq = q_ref[...] * inv
    std = jnp.sqrt(jnp.maximum(sq - mean * mean, 0.0) + 1e-8)
    agg = jnp.concatenate([mean, mx_ref[...], std], axis=1)
    out = (
        jnp.dot(agg, w2_ref[...], preferred_element_type=jnp.float32)
        + b2_ref[0:1, :]
    )
    out_ref[...] = out
    masked = out * nm_ref[...]
    colsum = jnp.sum(masked, axis=0, keepdims=True)
    colsq = jnp.sum(masked * out, axis=0, keepdims=True)
    part = jnp.concatenate(
        [jnp.broadcast_to(colsum, (4, D)), jnp.broadcast_to(colsq, (4, D))], axis=0
    )

    @pl.when(i == 0)
    def _():
        sums_ref[...] = jnp.zeros_like(sums_ref)

    sums_ref[...] += part


def _agg_matmul(s, q, mx, cnt, w2, b2, nmask):
    b22 = jnp.broadcast_to(b2.reshape(1, D), (8, D))
    return pl.pallas_call(
        _agg_body,
        grid=(N_PAD // NB,),
        in_specs=[
            pl.BlockSpec((NB, D), lambda i: (i, 0)),
            pl.BlockSpec((NB, D), lambda i: (i, 0)),
            pl.BlockSpec((NB, D), lambda i: (i, 0)),
            pl.BlockSpec((NB, 1), lambda i: (i, 0)),
            pl.BlockSpec((3 * D, D), lambda i: (0, 0)),
            pl.BlockSpec((8, D), lambda i: (0, 0)),
            pl.BlockSpec((NB, 1), lambda i: (i, 0)),
        ],
        out_specs=[
            pl.BlockSpec((NB, D), lambda i: (i, 0)),
            pl.BlockSpec((8, D), lambda i: (0, 0)),
        ],
        out_shape=[
            jax.ShapeDtypeStruct((N_PAD, D), jnp.float32),
            jax.ShapeDtypeStruct((8, D), jnp.float32),
        ],
    )(s, q, mx, cnt, w2, b22, nmask)


def _bn_res_body(o_ref, h_ref, sc_ref, sh_ref, out_ref):
    out_ref[...] = jnp.maximum(
        o_ref[...] * sc_ref[0:1, :] + sh_ref[0:1, :] + h_ref[...], 0.0
    )


def _bn_res(out_pre, h, scale, shift):
    sc = jnp.broadcast_to(scale.reshape(1, D), (8, D))
    sh = jnp.broadcast_to(shift.reshape(1, D), (8, D))
    return pl.pallas_call(
        _bn_res_body,
        grid=(N_PAD // NB,),
        in_specs=[
            pl.BlockSpec((NB, D), lambda i: (i, 0)),
            pl.BlockSpec((NB, D), lambda i: (i, 0)),
            pl.BlockSpec((8, D), lambda i: (0, 0)),
            pl.BlockSpec((8, D), lambda i: (0, 0)),
        ],
        out_specs=pl.BlockSpec((NB, D), lambda i: (i, 0)),
        out_shape=jax.ShapeDtypeStruct((N_PAD, D), jnp.float32),
    )(out_pre, h, sc, sh)


def kernel(x, edge_index, edge_attr, W1, b1, W2, b2, gamma, beta, W_out, b_out):
    n, d = x.shape
    e = edge_index.shape[1]
    src = edge_index[0].astype(jnp.int32)
    dst = edge_index[1].astype(jnp.int32)

    h = jnp.pad(x, ((0, N_PAD - n), (0, 0)))
    cnt = jax.ops.segment_sum(jnp.ones((e,), jnp.float32), dst, num_segments=N_PAD)
    cnt = jnp.maximum(cnt, 1.0).reshape(N_PAD, 1)
    nmask = (jnp.arange(N_PAD) < n).astype(jnp.float32).reshape(N_PAD, 1)

    nlayer = W1.shape[0]
    for l in range(nlayer):
        wab = jnp.concatenate([W1[l, :d], W1[l, d : 2 * d]], axis=1)
        bab = jnp.concatenate([jnp.zeros((d,), jnp.float32), b1[l]])
        ab = _matmul_bias(h, wab, bab, NB)
        a_proj = ab[:, :d]
        b_proj = ab[:, d:]
        c_proj = _matmul_bias(edge_attr, W1[l, 2 * d :], jnp.zeros((d,), jnp.float32), EB)

        m = jax.nn.relu(a_proj[src] + b_proj[dst] + c_proj)
        s = jax.ops.segment_sum(m, dst, num_segments=N_PAD)
        q = jax.ops.segment_sum(m * m, dst, num_segments=N_PAD)
        mx = jax.ops.segment_max(m, dst, num_segments=N_PAD)
        mx = jnp.where(jnp.isfinite(mx), mx, 0.0)

        out_pre, sums = _agg_matmul(s, q, mx, cnt, W2[l], b2[l], nmask)
        mu = sums[0] / n
        var = sums[4] / n - mu * mu
        scale = gamma[l] / jnp.sqrt(var + 1e-5)
        shift = beta[l] - mu * scale
        h = _bn_res(out_pre, h, scale, shift)

    out = _matmul_bias(h, W_out, b_out, NB)
    return out[:n]
